# baseline (speedup 1.0000x reference)
import functools

import jax
import jax.numpy as jnp
from jax.experimental import pallas as pl
from jax.experimental.pallas import tpu as pltpu
from jax.experimental.pallas import tpu_sc as plsc

_K = 8192
_D = 32
_TILE = 256
_KC = 4096
_GW = 128
_SC_CORES = 2
_SC_SUBCORES = 16


def _argmin_body(r_ref, rn2_ref, en2_ref, it_ref, et_ref, idx_ref, dmin_ref):
    r = r_ref[...]
    rn2 = rn2_ref[...]
    halves = []
    for h in range(2):
        best_d = None
        best_i = None
        for cc in range(_K // _KC // 2):
            c = h * (_K // _KC // 2) + cc
            et_c = et_ref[:, c * _KC:(c + 1) * _KC]
            s = jax.lax.dot_general(r.astype(jnp.bfloat16),
                                    et_c.astype(jnp.bfloat16),
                                    (((1,), (0,)), ((), ())),
                                    preferred_element_type=jnp.float32)
            en2_c = en2_ref[:, c * _KC:(c + 1) * _KC]
            dist = (rn2 - s) + en2_c
            m = jnp.min(dist, axis=1, keepdims=True)
            iota_c = it_ref[:, c * _KC:(c + 1) * _KC]
            ii = jnp.min(jnp.where(dist == m, iota_c, jnp.float32(_K)),
                         axis=1)
            md = m[:, 0]
            if best_d is None:
                best_d, best_i = md, ii
            else:
                take = md < best_d
                best_i = jnp.where(take, ii, best_i)
                best_d = jnp.where(take, md, best_d)
        halves.append((best_d, best_i))
    (d0, i0), (d1, i1) = halves
    acc = d0.astype(jnp.bfloat16).astype(jnp.float32)
    take = d1 < acc
    idx_ref[0, 0, :] = jnp.where(take, i1, i0).astype(jnp.int32)
    dmin_ref[0, 0, :] = jnp.where(take, d1, d0)


def _tc_argmin(r, rn2, en2, iota_row, et):
    rows = r.shape[0]
    grid = rows // _TILE
    outs = pl.pallas_call(
        _argmin_body,
        grid=(grid,),
        in_specs=[pl.BlockSpec((_TILE, _D), lambda i: (i, 0)),
                  pl.BlockSpec((_TILE, 1), lambda i: (i, 0)),
                  pl.BlockSpec((1, _K), lambda i: (0, 0)),
                  pl.BlockSpec((1, _K), lambda i: (0, 0)),
                  pl.BlockSpec((_D, _K), lambda i: (0, 0))],
        out_specs=[pl.BlockSpec((1, 1, _TILE), lambda i: (i, 0, 0)),
                   pl.BlockSpec((1, 1, _TILE), lambda i: (i, 0, 0))],
        out_shape=[jax.ShapeDtypeStruct((grid, 1, _TILE), jnp.int32),
                   jax.ShapeDtypeStruct((grid, 1, _TILE), jnp.float32)],
        compiler_params=pltpu.CompilerParams(
            dimension_semantics=("parallel",)),
    )(r, rn2, en2, iota_row, et)
    idx, dmin = outs
    return idx.reshape(rows), dmin


def _sc_gather(table, idx):
    n = idx.shape[0]
    nw = _SC_CORES * _SC_SUBCORES
    per_w = n // nw
    nch = per_w // _GW
    mesh = plsc.VectorSubcoreMesh(core_axis_name="core",
                                  subcore_axis_name="subcore")

    @functools.partial(
        pl.kernel,
        out_type=jax.ShapeDtypeStruct((n, 128), jnp.float32),
        mesh=mesh,
        scratch_types=[pltpu.VMEM((_GW,), jnp.int32),
                       pltpu.VMEM((_GW, 128), jnp.float32),
                       pltpu.SemaphoreType.DMA])
    def gather_kernel(table_hbm, i_hbm, o_hbm, idx_v, rows_v, sem):
        wid = (jax.lax.axis_index("subcore") * _SC_CORES
               + jax.lax.axis_index("core"))
        base = wid * per_w

        @pl.loop(0, nch)
        def _(c):
            off = base + c * _GW
            pltpu.sync_copy(i_hbm.at[pl.ds(off, _GW)], idx_v)
            pltpu.async_copy(table_hbm.at[idx_v], rows_v, sem).wait()
            pltpu.sync_copy(rows_v, o_hbm.at[pl.ds(off, _GW)])

    return gather_kernel(table, idx)


def _update_body(r_ref, q_ref, o_ref):
    o_ref[...] = r_ref[...] - q_ref[:, :_D]


def _tc_update(r, q):
    rows = r.shape[0]
    tile = 2048
    return pl.pallas_call(
        _update_body,
        grid=(rows // tile,),
        in_specs=[pl.BlockSpec((tile, _D), lambda i: (i, 0)),
                  pl.BlockSpec((tile, 128), lambda i: (i, 0))],
        out_specs=pl.BlockSpec((tile, _D), lambda i: (i, 0)),
        out_shape=jax.ShapeDtypeStruct((rows, _D), jnp.float32),
        compiler_params=pltpu.CompilerParams(
            dimension_semantics=("parallel",)),
    )(r, q)


def _combine_body(x_ref, r_ref, q_ref, o_ref):
    o_ref[...] = x_ref[...] - (r_ref[...] - q_ref[:, :_D])


def _tc_combine(x, r, q):
    rows = x.shape[0]
    tile = 2048
    spec = pl.BlockSpec((tile, _D), lambda i: (i, 0))
    qspec = pl.BlockSpec((tile, 128), lambda i: (i, 0))
    return pl.pallas_call(
        _combine_body,
        grid=(rows // tile,),
        in_specs=[spec, spec, qspec],
        out_specs=spec,
        out_shape=jax.ShapeDtypeStruct((rows, _D), jnp.float32),
        compiler_params=pltpu.CompilerParams(
            dimension_semantics=("parallel",)),
    )(x, r, q)


def kernel(x, codebooks):
    b, n, d = x.shape
    rows = b * n
    xf = x.reshape(rows, d)
    ets = jnp.transpose(codebooks, (0, 2, 1)) * 2.0
    cb_pad = jnp.pad(codebooks, ((0, 0), (0, 0), (0, 128 - d)))
    iota_row = jax.lax.iota(jnp.float32, _K).reshape(1, _K)
    num_q = codebooks.shape[0]

    idxs = []
    losses = []
    resid = xf
    quant = None
    for q in range(num_q):
        rn2 = jnp.sum(resid ** 2, axis=-1, keepdims=True)
        en2 = jnp.sum(codebooks[q] ** 2, axis=-1).reshape(1, _K)
        idx_q, dmin = _tc_argmin(resid, rn2, en2, iota_row, ets[q])
        quant = _sc_gather(cb_pad[q], idx_q)
        idxs.append(idx_q.reshape(b, n))
        losses.append(jnp.sum(dmin) / (rows * d))
        if q + 1 < num_q:
            resid = _tc_update(resid, quant)

    quantized_out = _tc_combine(xf, resid, quant).reshape(b, n, d)
    all_indices = jnp.stack(idxs, axis=-1)
    all_losses = jnp.stack(losses, axis=-1).astype(jnp.float32)
    return quantized_out, all_indices, all_losses


# speedup vs baseline: 1.0116x; 1.0116x over previous
import functools

import jax
import jax.numpy as jnp
from jax.experimental import pallas as pl
from jax.experimental.pallas import tpu as pltpu
from jax.experimental.pallas import tpu_sc as plsc

_K = 8192
_D = 32
_TILE = 256
_KC = 2048
_GW = 128
_SC_CORES = 2
_SC_SUBCORES = 16


def _argmin_body(r_ref, rn2_ref, en2_ref, it_ref, et_ref, idx_ref, dmin_ref):
    r = r_ref[...]
    rn2 = rn2_ref[...]
    halves = []
    for h in range(2):
        best_d = None
        best_i = None
        for cc in range(_K // _KC // 2):
            c = h * (_K // _KC // 2) + cc
            et_c = et_ref[:, c * _KC:(c + 1) * _KC]
            s = jax.lax.dot_general(r.astype(jnp.bfloat16),
                                    et_c.astype(jnp.bfloat16),
                                    (((1,), (0,)), ((), ())),
                                    preferred_element_type=jnp.float32)
            en2_c = en2_ref[:, c * _KC:(c + 1) * _KC]
            dist = (rn2 - s) + en2_c
            m = jnp.min(dist, axis=1, keepdims=True)
            iota_c = it_ref[:, c * _KC:(c + 1) * _KC]
            ii = jnp.min(jnp.where(dist == m, iota_c, jnp.float32(_K)),
                         axis=1)
            md = m[:, 0]
            if best_d is None:
                best_d, best_i = md, ii
            else:
                take = md < best_d
                best_i = jnp.where(take, ii, best_i)
                best_d = jnp.where(take, md, best_d)
        halves.append((best_d, best_i))
    (d0, i0), (d1, i1) = halves
    acc = d0.astype(jnp.bfloat16).astype(jnp.float32)
    take = d1 < acc
    idx_ref[0, 0, :] = jnp.where(take, i1, i0).astype(jnp.int32)
    dmin_ref[0, 0, :] = jnp.where(take, d1, d0)


def _tc_argmin(r, rn2, en2, iota_row, et):
    rows = r.shape[0]
    grid = rows // _TILE
    outs = pl.pallas_call(
        _argmin_body,
        grid=(grid,),
        in_specs=[pl.BlockSpec((_TILE, _D), lambda i: (i, 0)),
                  pl.BlockSpec((_TILE, 1), lambda i: (i, 0)),
                  pl.BlockSpec((1, _K), lambda i: (0, 0)),
                  pl.BlockSpec((1, _K), lambda i: (0, 0)),
                  pl.BlockSpec((_D, _K), lambda i: (0, 0))],
        out_specs=[pl.BlockSpec((1, 1, _TILE), lambda i: (i, 0, 0)),
                   pl.BlockSpec((1, 1, _TILE), lambda i: (i, 0, 0))],
        out_shape=[jax.ShapeDtypeStruct((grid, 1, _TILE), jnp.int32),
                   jax.ShapeDtypeStruct((grid, 1, _TILE), jnp.float32)],
        compiler_params=pltpu.CompilerParams(
            dimension_semantics=("parallel",)),
    )(r, rn2, en2, iota_row, et)
    idx, dmin = outs
    return idx.reshape(rows), dmin


def _sc_gather(table, idx):
    n = idx.shape[0]
    nw = _SC_CORES * _SC_SUBCORES
    per_w = n // nw
    nch = per_w // _GW
    mesh = plsc.VectorSubcoreMesh(core_axis_name="core",
                                  subcore_axis_name="subcore")

    @functools.partial(
        pl.kernel,
        out_type=jax.ShapeDtypeStruct((n, 128), jnp.float32),
        mesh=mesh,
        scratch_types=[pltpu.VMEM((_GW,), jnp.int32),
                       pltpu.VMEM((_GW, 128), jnp.float32),
                       pltpu.SemaphoreType.DMA])
    def gather_kernel(table_hbm, i_hbm, o_hbm, idx_v, rows_v, sem):
        wid = (jax.lax.axis_index("subcore") * _SC_CORES
               + jax.lax.axis_index("core"))
        base = wid * per_w

        @pl.loop(0, nch)
        def _(c):
            off = base + c * _GW
            pltpu.sync_copy(i_hbm.at[pl.ds(off, _GW)], idx_v)
            pltpu.async_copy(table_hbm.at[idx_v], rows_v, sem).wait()
            pltpu.sync_copy(rows_v, o_hbm.at[pl.ds(off, _GW)])

    return gather_kernel(table, idx)


def _update_body(r_ref, q_ref, o_ref):
    o_ref[...] = r_ref[...] - q_ref[:, :_D]


def _tc_update(r, q):
    rows = r.shape[0]
    tile = 2048
    return pl.pallas_call(
        _update_body,
        grid=(rows // tile,),
        in_specs=[pl.BlockSpec((tile, _D), lambda i: (i, 0)),
                  pl.BlockSpec((tile, 128), lambda i: (i, 0))],
        out_specs=pl.BlockSpec((tile, _D), lambda i: (i, 0)),
        out_shape=jax.ShapeDtypeStruct((rows, _D), jnp.float32),
        compiler_params=pltpu.CompilerParams(
            dimension_semantics=("parallel",)),
    )(r, q)


def _combine_body(x_ref, r_ref, q_ref, o_ref):
    o_ref[...] = x_ref[...] - (r_ref[...] - q_ref[:, :_D])


def _tc_combine(x, r, q):
    rows = x.shape[0]
    tile = 2048
    spec = pl.BlockSpec((tile, _D), lambda i: (i, 0))
    qspec = pl.BlockSpec((tile, 128), lambda i: (i, 0))
    return pl.pallas_call(
        _combine_body,
        grid=(rows // tile,),
        in_specs=[spec, spec, qspec],
        out_specs=spec,
        out_shape=jax.ShapeDtypeStruct((rows, _D), jnp.float32),
        compiler_params=pltpu.CompilerParams(
            dimension_semantics=("parallel",)),
    )(x, r, q)


def kernel(x, codebooks):
    b, n, d = x.shape
    rows = b * n
    xf = x.reshape(rows, d)
    ets = jnp.transpose(codebooks, (0, 2, 1)) * 2.0
    cb_pad = jnp.pad(codebooks, ((0, 0), (0, 0), (0, 128 - d)))
    iota_row = jax.lax.iota(jnp.float32, _K).reshape(1, _K)
    num_q = codebooks.shape[0]

    idxs = []
    losses = []
    resid = xf
    quant = None
    for q in range(num_q):
        rn2 = jnp.sum(resid ** 2, axis=-1, keepdims=True)
        en2 = jnp.sum(codebooks[q] ** 2, axis=-1).reshape(1, _K)
        idx_q, dmin = _tc_argmin(resid, rn2, en2, iota_row, ets[q])
        quant = _sc_gather(cb_pad[q], idx_q)
        idxs.append(idx_q.reshape(b, n))
        losses.append(jnp.sum(dmin) / (rows * d))
        if q + 1 < num_q:
            resid = _tc_update(resid, quant)

    quantized_out = _tc_combine(xf, resid, quant).reshape(b, n, d)
    all_indices = jnp.stack(idxs, axis=-1)
    all_losses = jnp.stack(losses, axis=-1).astype(jnp.float32)
    return quantized_out, all_indices, all_losses
